# Initial kernel scaffold; baseline (speedup 1.0000x reference)
import jax
import jax.numpy as jnp
from jax.experimental import pallas as pl


def kernel(x, edge_index, W1, b1, W2, b2):
    raise NotImplementedError("write your pallas kernel here")


# speedup vs baseline: 10.3787x; 10.3787x over previous
import functools

import jax
import jax.numpy as jnp
from jax import lax
from jax.experimental import pallas as pl
from jax.experimental.pallas import tpu as pltpu
from jax.experimental.pallas import tpu_sc as plsc

NC = 2
NS = 16
NW = NC * NS
WIN = 128
BR = 512
SCL = 16


def _sc_mesh():
    return plsc.VectorSubcoreMesh(core_axis_name="c", subcore_axis_name="s")


ZR = 16


def _zero_fill(ref, rows, cols):
    @pl.loop(0, rows)
    def _(i):
        @pl.loop(0, cols // SCL)
        def _(j):
            ref.at[pl.ds(i, 1), pl.ds(j * SCL, SCL)][...] = jnp.zeros(
                (1, SCL), jnp.float32)


def _deg_call(dst_w, n_pad):
    num_win = dst_w.shape[0]
    wpw = num_win // NW
    tile_rows = n_pad // NS

    @functools.partial(
        pl.kernel,
        out_type=jax.ShapeDtypeStruct((NC, n_pad, SCL), jnp.float32),
        mesh=_sc_mesh(),
        scratch_types=[
            pltpu.VMEM_SHARED((n_pad, SCL), jnp.float32),
            pltpu.VMEM((WIN, SCL), jnp.float32),
            pltpu.VMEM((ZR, SCL), jnp.float32),
            pltpu.VMEM((WIN,), jnp.int32),
        ],
    )
    def k(dst_hbm, out_hbm, acc, ones_v, zb, dst_v):
        c = lax.axis_index("c")
        s = lax.axis_index("s")
        w = c * NS + s
        base = s * tile_rows

        @pl.loop(0, WIN)
        def _(i):
            ones_v.at[pl.ds(i, 1), pl.ds(0, SCL)][...] = jnp.ones(
                (1, SCL), jnp.float32)

        _zero_fill(zb, ZR, SCL)

        @pl.loop(0, tile_rows // ZR)
        def _(b):
            pltpu.sync_copy(zb, acc.at[pl.ds(base + b * ZR, ZR)])

        plsc.subcore_barrier()

        @pl.loop(0, wpw)
        def _(j):
            win = w * wpw + j
            pltpu.sync_copy(dst_hbm.at[win], dst_v)
            pltpu.sync_copy(ones_v, acc.at[dst_v], add=True)

        plsc.subcore_barrier()
        pltpu.sync_copy(acc.at[pl.ds(base, tile_rows)],
                        out_hbm.at[c].at[pl.ds(base, tile_rows)])

    return k(dst_w)


def _agg_call(u_pad, src_w, dst_w):
    n_pad, d = u_pad.shape
    num_win = src_w.shape[0]
    wpw = num_win // NW
    tile_rows = n_pad // NS

    @functools.partial(
        pl.kernel,
        out_type=jax.ShapeDtypeStruct((NC, n_pad, d), jnp.float32),
        mesh=_sc_mesh(),
        scratch_types=[
            pltpu.VMEM_SHARED((n_pad, d), jnp.float32),
            pltpu.VMEM((WIN, d), jnp.float32),
            pltpu.VMEM((ZR, d), jnp.float32),
            pltpu.VMEM((WIN,), jnp.int32),
            pltpu.VMEM((WIN,), jnp.int32),
            pltpu.SemaphoreType.DMA,
        ],
    )
    def k(u_hbm, src_hbm, dst_hbm, out_hbm, acc, rows_v, zb, src_v, dst_v,
          sem):
        c = lax.axis_index("c")
        s = lax.axis_index("s")
        w = c * NS + s
        base = s * tile_rows

        _zero_fill(zb, ZR, d)

        @pl.loop(0, tile_rows // ZR)
        def _(b):
            pltpu.sync_copy(zb, acc.at[pl.ds(base + b * ZR, ZR)])

        plsc.subcore_barrier()

        @pl.loop(0, wpw)
        def _(j):
            win = w * wpw + j
            pltpu.sync_copy(src_hbm.at[win], src_v)
            pltpu.sync_copy(dst_hbm.at[win], dst_v)
            pltpu.async_copy(u_hbm.at[src_v], rows_v, sem).wait()
            pltpu.sync_copy(rows_v, acc.at[dst_v], add=True)

        plsc.subcore_barrier()
        pltpu.sync_copy(acc.at[pl.ds(base, tile_rows)],
                        out_hbm.at[c].at[pl.ds(base, tile_rows)])

    return k(u_pad, src_w, dst_w)


def _mm_body(x_ref, w_ref, o_ref):
    o_ref[...] = lax.dot_general(
        x_ref[...], w_ref[...], (((1,), (0,)), ((), ())),
        preferred_element_type=jnp.float32, precision=lax.Precision.HIGHEST)


def _finish1_body(h_ref, da_ref, db_ref, u_ref, dinv_ref):
    cnt = da_ref[:, 0:1] + db_ref[:, 0:1]
    dinv = lax.rsqrt(cnt + 1.0)
    dinv_ref[...] = dinv
    u_ref[...] = h_ref[...] * dinv


def _mid_body(aa_ref, ab_ref, u1_ref, dinv_ref, b1_ref, w2_ref, u2_ref):
    dinv = dinv_ref[...]
    z = (aa_ref[...] + ab_ref[...] + u1_ref[...]) * dinv + b1_ref[...]
    z = jnp.maximum(z, 0.0)
    h2 = lax.dot_general(
        z, w2_ref[...], (((1,), (0,)), ((), ())),
        preferred_element_type=jnp.float32, precision=lax.Precision.HIGHEST)
    u2_ref[...] = h2 * dinv


def _out_body(aa_ref, ab_ref, u2_ref, dinv_ref, b2_ref, o_ref):
    o_ref[...] = ((aa_ref[...] + ab_ref[...] + u2_ref[...]) * dinv_ref[...]
                  + b2_ref[...])


def _rows_spec(d):
    return pl.BlockSpec((BR, d), lambda i: (i, 0))


def _full_spec(r, c):
    return pl.BlockSpec((r, c), lambda i: (0, 0))


@jax.jit
def _run(x, edge_index, W1, b1, W2, b2):
    n, d = x.shape
    e = edge_index.shape[1]
    n_pad = ((n + 1 + NS * ZR - 1) // (NS * ZR)) * (NS * ZR)
    e_pad = ((e + NW * WIN - 1) // (NW * WIN)) * (NW * WIN)

    src = edge_index[0].astype(jnp.int32)
    dst = edge_index[1].astype(jnp.int32)
    pad_idx = jnp.full((e_pad - e,), n, jnp.int32)
    src_w = jnp.concatenate([src, pad_idx]).reshape(-1, WIN)
    dst_w = jnp.concatenate([dst, pad_idx]).reshape(-1, WIN)
    x_pad = jnp.pad(x, ((0, n_pad - n), (0, 0)))

    grid = (n_pad // BR,)

    deg = _deg_call(dst_w, n_pad)

    h1 = pl.pallas_call(
        _mm_body, grid=grid,
        in_specs=[_rows_spec(d), _full_spec(d, d)],
        out_specs=_rows_spec(d),
        out_shape=jax.ShapeDtypeStruct((n_pad, d), jnp.float32),
    )(x_pad, W1)

    u1, dinv = pl.pallas_call(
        _finish1_body, grid=grid,
        in_specs=[_rows_spec(d), _rows_spec(SCL), _rows_spec(SCL)],
        out_specs=[_rows_spec(d), _rows_spec(1)],
        out_shape=[jax.ShapeDtypeStruct((n_pad, d), jnp.float32),
                   jax.ShapeDtypeStruct((n_pad, 1), jnp.float32)],
    )(h1, deg[0], deg[1])

    agg1 = _agg_call(u1, src_w, dst_w)

    u2 = pl.pallas_call(
        _mid_body, grid=grid,
        in_specs=[_rows_spec(d), _rows_spec(d), _rows_spec(d), _rows_spec(1),
                  _full_spec(1, d), _full_spec(d, d)],
        out_specs=_rows_spec(d),
        out_shape=jax.ShapeDtypeStruct((n_pad, d), jnp.float32),
    )(agg1[0], agg1[1], u1, dinv, b1.reshape(1, d), W2)

    agg2 = _agg_call(u2, src_w, dst_w)

    out = pl.pallas_call(
        _out_body, grid=grid,
        in_specs=[_rows_spec(d), _rows_spec(d), _rows_spec(d), _rows_spec(1),
                  _full_spec(1, d)],
        out_specs=_rows_spec(d),
        out_shape=jax.ShapeDtypeStruct((n_pad, d), jnp.float32),
    )(agg2[0], agg2[1], u2, dinv, b2.reshape(1, d))

    return out[:n]


def kernel(x, edge_index, W1, b1, W2, b2):
    return _run(x, edge_index, W1, b1, W2, b2)
